# baseline (speedup 1.0000x reference)
import jax
import jax.numpy as jnp
from jax.experimental import pallas as pl

_K = 20
_EPS = 1e-5


def _knn_idx(x, k):
    inner = -2.0 * jnp.matmul(jnp.transpose(x, (0, 2, 1)), x)
    xx = jnp.sum(x ** 2, axis=1, keepdims=True)
    pd = -xx - inner - jnp.transpose(xx, (0, 2, 1))
    return jax.lax.top_k(pd, k)[1]


def _gf(x, k):
    B, C, N = x.shape
    idx = _knn_idx(x, k)
    xt = jnp.transpose(x, (0, 2, 1))
    gathered = jax.vmap(lambda pts, ind: pts[ind])(xt, idx)
    center = jnp.broadcast_to(xt[:, :, None, :], (B, N, k, C))
    feat = jnp.concatenate([gathered - center, center], axis=3)
    return jnp.transpose(feat, (0, 3, 1, 2))


def _norm(x, g, b):
    axes = tuple(i for i in range(x.ndim) if i != 1)
    mean = jnp.mean(x, axis=axes, keepdims=True)
    var = jnp.var(x, axis=axes, keepdims=True)
    shp = [1] * x.ndim
    shp[1] = -1
    return g.reshape(shp) * (x - mean) / jnp.sqrt(var + _EPS) + b.reshape(shp)


def _lr(x):
    return jnp.where(x >= 0, x, 0.2 * x)


def _final_lrelu_kernel(x_ref, o_ref):
    v = x_ref[...]
    o_ref[...] = jnp.where(v >= 0, v, 0.2 * v)


def kernel(x, conv1_w, conv2_w, conv3_w, conv4_w, bn1_g, bn1_b, bn2_g, bn2_b, bn3_g, bn3_b, bn4_g, bn4_b):
    x1 = x
    h = _gf(x1, _K)
    h = _lr(_norm(jnp.einsum('oc,bcnk->bonk', conv1_w, h), bn1_g, bn1_b))
    h = _lr(_norm(jnp.einsum('oc,bcnk->bonk', conv2_w, h), bn2_g, bn2_b))
    x2 = jnp.max(h, axis=-1)
    h = _gf(x2, _K)
    h = _lr(_norm(jnp.einsum('oc,bcnk->bonk', conv3_w, h), bn3_g, bn3_b))
    x3 = jnp.max(h, axis=-1)
    cat = jnp.concatenate([x1, x2, x3], axis=1)
    out = _norm(jnp.einsum('oc,bcn->bon', conv4_w, cat), bn4_g, bn4_b)
    B, C, N = out.shape
    out = pl.pallas_call(
        _final_lrelu_kernel,
        out_shape=jax.ShapeDtypeStruct((B, C, N), out.dtype),
        grid=(B,),
        in_specs=[pl.BlockSpec((1, C, N), lambda b: (b, 0, 0))],
        out_specs=pl.BlockSpec((1, C, N), lambda b: (b, 0, 0)),
    )(out)
    return out


# speedup vs baseline: 9.5377x; 9.5377x over previous
import functools

import jax
import jax.numpy as jnp
from jax import lax
from jax.experimental import pallas as pl
from jax.experimental.pallas import tpu as pltpu
from jax.experimental.pallas import tpu_sc as plsc

_K = 20
_EPS = 1e-5
_B = 8
_N = 2048
_C = 64
_R = 512
_PB = 128
_RB = _PB * _K
_BN = _B * _N
_BNK = _BN * _K
_NBLK = _BNK // _RB
_NW = 32
_SCROWS = _BNK // _NW
_SCCH = 256
_SCG = _SCCH // 128
_TW = 128

_NT = (((1,), (1,)), ((), ()))
_NN = (((1,), (0,)), ((), ()))


def _lrelu(v):
    return jnp.where(v >= 0, v, 0.2 * v)


def _bn_apply(u, m_ref, v_ref, g_ref, b_ref):
    return (g_ref[...] * (u - m_ref[...]) / jnp.sqrt(v_ref[...] + _EPS)
            + b_ref[...])


def _knn_body(xt_ref, x_ref, idx_ref):
    b = pl.program_id(0)
    xt_r = xt_ref[0]
    xf = x_ref[0]
    inner = lax.dot_general(xt_r, xf, _NN, preferred_element_type=jnp.float32)
    xx_r = jnp.sum(xt_r * xt_r, axis=1, keepdims=True)
    xx_c = jnp.sum(xf * xf, axis=0, keepdims=True)
    m = 2.0 * inner - xx_r - xx_c
    iota = lax.broadcasted_iota(jnp.int32, (_R, _N), 1)
    neg = jnp.float32(-jnp.inf)
    cols = []
    for _ in range(_K):
        mx = jnp.max(m, axis=1, keepdims=True)
        cand = jnp.where(m == mx, iota, _N)
        amin = jnp.min(cand, axis=1, keepdims=True)
        cols.append(amin)
        m = jnp.where(iota == amin, neg, m)
    idx_ref[0] = jnp.concatenate(cols, axis=1) + b * _N


def _knn(xt, x):
    return pl.pallas_call(
        _knn_body,
        grid=(_B, _N // _R),
        in_specs=[
            pl.BlockSpec((1, _R, _C), lambda b, i: (b, i, 0)),
            pl.BlockSpec((1, _C, _N), lambda b, i: (b, 0, 0)),
        ],
        out_specs=pl.BlockSpec((1, _R, _K), lambda b, i: (b, i, 0)),
        out_shape=jax.ShapeDtypeStruct((_B, _N, _K), jnp.int32),
    )(xt, x)


def _sc_gather(table, idx2d):
    mesh = plsc.VectorSubcoreMesh(core_axis_name="c", subcore_axis_name="s")
    nir = _SCROWS // 128
    nch = _SCROWS // _SCCH

    @functools.partial(
        pl.kernel,
        out_type=jax.ShapeDtypeStruct((_BNK, _TW), jnp.float32),
        mesh=mesh,
        scratch_types=[
            pltpu.VMEM((nir, 128), jnp.int32),
            pltpu.VMEM((_SCCH, _TW), jnp.float32),
            pltpu.VMEM((_SCCH, _TW), jnp.float32),
            pltpu.SemaphoreType.DMA,
            pltpu.SemaphoreType.DMA,
            pltpu.SemaphoreType.DMA,
            pltpu.SemaphoreType.DMA,
        ],
    )
    def k(table_hbm, idx_hbm, out_hbm, idx_v, buf0, buf1, g0, g1, s0, s1):
        wid = lax.axis_index("c") * 16 + lax.axis_index("s")
        pltpu.sync_copy(idx_hbm.at[pl.ds(wid * nir, nir)], idx_v)
        bufs = (buf0, buf1)
        gsems = (g0, g1)
        ssems = (s0, s1)
        store_h = [None, None]
        for c in range(nch):
            cb = c & 1
            if store_h[cb] is not None:
                store_h[cb].wait()
            hs = []
            for j in range(_SCG):
                hs.append(pltpu.async_copy(
                    table_hbm.at[idx_v.at[c * _SCG + j]],
                    bufs[cb].at[pl.ds(j * 128, 128)],
                    gsems[cb]))
            for h in hs:
                h.wait()
            store_h[cb] = pltpu.async_copy(
                bufs[cb],
                out_hbm.at[pl.ds(wid * _SCROWS + c * _SCCH, _SCCH)],
                ssems[cb])
        for h in store_h:
            if h is not None:
                h.wait()

    return k(table, idx2d)


def _econv_body(e_ref, c_ref, w_ref, u_ref, o_ref):
    i = pl.program_id(0)
    g = e_ref[0][:, :_C].reshape(_PB, _K, _C)
    cen = c_ref[0]
    gc = (g - cen[:, None, :]).reshape(_RB, _C)
    cr = jnp.broadcast_to(cen[:, None, :], (_PB, _K, _C)).reshape(_RB, _C)
    feat = jnp.concatenate([gc, cr], axis=1)
    u = lax.dot_general(feat, w_ref[...], _NT,
                        preferred_element_type=jnp.float32)
    u_ref[0] = u
    st = jnp.stack([jnp.sum(u, axis=0), jnp.sum(u * u, axis=0)])

    @pl.when(i == 0)
    def _():
        o_ref[...] = jnp.zeros_like(o_ref)

    o_ref[...] += st


def _econv(e, xt, w):
    return pl.pallas_call(
        _econv_body,
        grid=(_NBLK,),
        in_specs=[
            pl.BlockSpec((1, _RB, _TW), lambda i: (i, 0, 0)),
            pl.BlockSpec((1, _PB, _C), lambda i: (i, 0, 0)),
            pl.BlockSpec((_C, 2 * _C), lambda i: (0, 0)),
        ],
        out_specs=[
            pl.BlockSpec((1, _RB, _C), lambda i: (i, 0, 0)),
            pl.BlockSpec((2, _C), lambda i: (0, 0)),
        ],
        out_shape=[
            jax.ShapeDtypeStruct((_NBLK, _RB, _C), jnp.float32),
            jax.ShapeDtypeStruct((2, _C), jnp.float32),
        ],
    )(e.reshape(_NBLK, _RB, _TW), xt.reshape(_NBLK, _PB, _C), w)


def _conv2_body(u_ref, m_ref, v_ref, g_ref, b_ref, w_ref, u2_ref, o_ref):
    i = pl.program_id(0)
    t = _lrelu(_bn_apply(u_ref[0], m_ref, v_ref, g_ref, b_ref))
    u2 = lax.dot_general(t, w_ref[...], _NT,
                         preferred_element_type=jnp.float32)
    u2_ref[0] = u2
    st = jnp.stack([jnp.sum(u2, axis=0), jnp.sum(u2 * u2, axis=0)])

    @pl.when(i == 0)
    def _():
        o_ref[...] = jnp.zeros_like(o_ref)

    o_ref[...] += st


def _conv2_pass(u, mv, g, b, w2):
    sspec = pl.BlockSpec((1, _C), lambda i: (0, 0))
    return pl.pallas_call(
        _conv2_body,
        grid=(_NBLK,),
        in_specs=[
            pl.BlockSpec((1, _RB, _C), lambda i: (i, 0, 0)),
            sspec, sspec, sspec, sspec,
            pl.BlockSpec((_C, _C), lambda i: (0, 0)),
        ],
        out_specs=[
            pl.BlockSpec((1, _RB, _C), lambda i: (i, 0, 0)),
            pl.BlockSpec((2, _C), lambda i: (0, 0)),
        ],
        out_shape=[
            jax.ShapeDtypeStruct((_NBLK, _RB, _C), jnp.float32),
            jax.ShapeDtypeStruct((2, _C), jnp.float32),
        ],
    )(u, mv[0], mv[1], g.reshape(1, _C), b.reshape(1, _C), w2)


def _max_body(u_ref, m_ref, v_ref, g_ref, b_ref, xt_ref, x_ref):
    t = _lrelu(_bn_apply(u_ref[0], m_ref, v_ref, g_ref, b_ref))
    r = jnp.max(t.reshape(_PB, _K, _C), axis=1)
    xt_ref[0] = r
    x_ref[0] = r.T


def _max_pass(u, mv, g, b):
    nb = _N // _PB
    sspec = pl.BlockSpec((1, _C), lambda i: (0, 0))
    return pl.pallas_call(
        _max_body,
        grid=(_NBLK,),
        in_specs=[
            pl.BlockSpec((1, _RB, _C), lambda i: (i, 0, 0)),
            sspec, sspec, sspec, sspec,
        ],
        out_specs=[
            pl.BlockSpec((1, _PB, _C), lambda i: (i, 0, 0)),
            pl.BlockSpec((1, _C, _PB), lambda i: (i // nb, 0, i % nb)),
        ],
        out_shape=[
            jax.ShapeDtypeStruct((_NBLK, _PB, _C), jnp.float32),
            jax.ShapeDtypeStruct((_B, _C, _N), jnp.float32),
        ],
    )(u, mv[0], mv[1], g.reshape(1, _C), b.reshape(1, _C))


def _norm_max(u, mv, g, b):
    sspec = pl.BlockSpec((1, _C), lambda i: (0, 0))

    def body(u_ref, m_ref, v_ref, g_ref, b_ref, xt_ref):
        t = _lrelu(_bn_apply(u_ref[0], m_ref, v_ref, g_ref, b_ref))
        xt_ref[0] = jnp.max(t.reshape(_PB, _K, _C), axis=1)

    return pl.pallas_call(
        body,
        grid=(_NBLK,),
        in_specs=[
            pl.BlockSpec((1, _RB, _C), lambda i: (i, 0, 0)),
            sspec, sspec, sspec, sspec,
        ],
        out_specs=pl.BlockSpec((1, _PB, _C), lambda i: (i, 0, 0)),
        out_shape=jax.ShapeDtypeStruct((_NBLK, _PB, _C), jnp.float32),
    )(u, mv[0], mv[1], g.reshape(1, _C), b.reshape(1, _C))


def _c4_stats_body(x1_ref, x2_ref, x3_ref, w_ref, o_ref):
    cat = jnp.concatenate([x1_ref[0], x2_ref[0], x3_ref[0]], axis=1)
    y = lax.dot_general(cat, w_ref[...], _NT,
                        preferred_element_type=jnp.float32)
    st = jnp.stack([jnp.sum(y, axis=0), jnp.sum(y * y, axis=0)])

    @pl.when(jnp.logical_and(pl.program_id(0) == 0, pl.program_id(1) == 0))
    def _():
        o_ref[...] = jnp.zeros_like(o_ref)

    o_ref[...] += st


def _c4_out_body(x1_ref, x2_ref, x3_ref, w_ref, m_ref, v_ref, g_ref, b_ref,
                 o_ref):
    cat = jnp.concatenate([x1_ref[0], x2_ref[0], x3_ref[0]], axis=1)
    y = lax.dot_general(w_ref[...], cat, _NT,
                        preferred_element_type=jnp.float32)
    cm = lambda r: r[...].reshape(-1, 1)
    o_ref[0] = _lrelu(cm(g_ref) * (y - cm(m_ref)) / jnp.sqrt(cm(v_ref) + _EPS)
                      + cm(b_ref))


def _conv4(xt, x2t, x3t, w4, g4, b4):
    co = w4.shape[0]
    row_specs = [pl.BlockSpec((1, _R, _C), lambda b, i: (b, i, 0))] * 3
    wspec = pl.BlockSpec((co, 3 * _C), lambda b, i: (0, 0))
    st = pl.pallas_call(
        _c4_stats_body,
        grid=(_B, _N // _R),
        in_specs=row_specs + [wspec],
        out_specs=pl.BlockSpec((2, co), lambda b, i: (0, 0)),
        out_shape=jax.ShapeDtypeStruct((2, co), jnp.float32),
    )(xt, x2t, x3t, w4)
    mv = _derive(st, _BN)
    sspec = pl.BlockSpec((1, co), lambda b, i: (0, 0))
    return pl.pallas_call(
        _c4_out_body,
        grid=(_B, _N // _R),
        in_specs=row_specs + [wspec, sspec, sspec, sspec, sspec],
        out_specs=pl.BlockSpec((1, co, _R), lambda b, i: (b, 0, i)),
        out_shape=jax.ShapeDtypeStruct((_B, co, _N), jnp.float32),
    )(xt, x2t, x3t, w4, mv[0], mv[1], g4.reshape(1, co), b4.reshape(1, co))


def _derive(st, cnt):
    mean = st[0] / cnt
    var = st[1] / cnt - mean * mean
    return mean.reshape(1, -1), var.reshape(1, -1)


def _pad_table(xt):
    return jnp.concatenate(
        [xt.reshape(_BN, _C), jnp.zeros((_BN, _TW - _C), jnp.float32)], axis=1)


def kernel(x, conv1_w, conv2_w, conv3_w, conv4_w, bn1_g, bn1_b, bn2_g, bn2_b,
           bn3_g, bn3_b, bn4_g, bn4_b):
    xt = jnp.transpose(x, (0, 2, 1))

    idx1 = _knn(xt, x)
    e1 = _sc_gather(_pad_table(xt), idx1.reshape(_BNK // 128, 128))
    u1, st1 = _econv(e1, xt, conv1_w)
    u2, st2 = _conv2_pass(u1, _derive(st1, _BNK), bn1_g, bn1_b, conv2_w)
    x2t, x2 = _max_pass(u2, _derive(st2, _BNK), bn2_g, bn2_b)
    x2t = x2t.reshape(_B, _N, _C)

    idx2 = _knn(x2t, x2)
    e2 = _sc_gather(_pad_table(x2t), idx2.reshape(_BNK // 128, 128))
    u3, st3 = _econv(e2, x2t, conv3_w)
    x3t = _norm_max(u3, _derive(st3, _BNK), bn3_g, bn3_b).reshape(_B, _N, _C)

    return _conv4(xt, x2t, x3t, conv4_w, bn4_g, bn4_b)
